# baseline (speedup 1.0000x reference)
import jax
import jax.numpy as jnp
from jax.experimental import pallas as pl
from jax.experimental.pallas import tpu as pltpu

_CB = 128


def _css_body(p_ref, fa_ref, fb_ref, oa_ref, ob_ref, w_ref):
    cb = pl.program_id(1)

    @pl.when(cb == 0)
    def _():
        p = p_ref[0]
        m = jnp.max(p, axis=1, keepdims=True)
        e = jnp.exp(p - m)
        z = jnp.sum(e, axis=1, keepdims=True)
        w_ref[...] = e * (1.0 / z)

    w = w_ref[...]
    oa_ref[0, 0] = jax.lax.dot_general(
        fa_ref[0], w, (((1,), (1,)), ((), ())),
        preferred_element_type=jnp.float32)
    ob_ref[0, 0] = jax.lax.dot_general(
        fb_ref[0], w, (((1,), (1,)), ((), ())),
        preferred_element_type=jnp.float32)


def kernel(feats, probs):
    B, K, H, W = probs.shape
    C = feats.shape[1]
    HW = H * W
    f = feats.reshape(B, C, HW)
    p = probs.reshape(B, K, HW)
    npair = C // (2 * _CB)
    outs = pl.pallas_call(
        _css_body,
        grid=(B, npair),
        in_specs=[
            pl.BlockSpec((1, K, HW), lambda b, cb: (b, 0, 0)),
            pl.BlockSpec((1, _CB, HW), lambda b, cb: (b, 2 * cb, 0)),
            pl.BlockSpec((1, _CB, HW), lambda b, cb: (b, 2 * cb + 1, 0)),
        ],
        out_specs=[
            pl.BlockSpec((1, 1, _CB, K), lambda b, cb: (b, cb, 0, 0)),
            pl.BlockSpec((1, 1, _CB, K), lambda b, cb: (b, cb, 0, 0)),
        ],
        out_shape=[
            jax.ShapeDtypeStruct((B, npair, _CB, K), jnp.float32),
            jax.ShapeDtypeStruct((B, npair, _CB, K), jnp.float32),
        ],
        scratch_shapes=[
            pltpu.VMEM((K, HW), jnp.float32),
        ],
        compiler_params=pltpu.CompilerParams(
            dimension_semantics=("parallel", "arbitrary"),
            vmem_limit_bytes=60 * 1024 * 1024,
        ),
        name="css_softmax_pool",
    )(p, f, f)
    oa, ob = outs
    out = jnp.stack([oa, ob], axis=2).reshape(B, C, K)
    return out[..., None]
